# baseline (speedup 1.0000x reference)
import functools

import jax
import jax.numpy as jnp
from jax import lax
from jax.experimental import pallas as pl
from jax.experimental.pallas import tpu as pltpu
from jax.experimental.pallas import tpu_sc as plsc

PRE_LEN = 128
B_ROWS = 1024
D = 49152
NCH = 32
DC = D // NCH
R = B_ROWS * NCH
NW = 32
BPW = B_ROWS // NW
G = 16
NGRP = BPW // G


def _gather_kernel(t_hbm, idx_hbm, out_hbm, idx_full, gstage, ostage, bufs,
                   sem_g, sem_s):
    wid = lax.axis_index("s") * 2 + lax.axis_index("c")
    bs0 = wid * BPW
    pltpu.sync_copy(idx_hbm.at[pl.ds(bs0, BPW)], idx_full)
    lanes = lax.iota(jnp.int32, G)

    def body(it, carry):
        c = lax.shift_right_logical(it, 1)
        grp = lax.bitwise_and(it, 1)
        pv = idx_full[pl.ds(grp * G, G)]
        gstage[...] = pv * NCH + c
        ostage[...] = (bs0 + grp * G + lanes) * NCH + c
        cp_g = pltpu.make_async_copy(t_hbm.at[gstage], bufs, sem_g)
        cp_g.start()
        cp_g.wait()
        cp_s = pltpu.make_async_copy(bufs, out_hbm.at[ostage], sem_s)
        cp_s.start()
        cp_s.wait()
        return carry

    lax.fori_loop(0, NCH * NGRP, body, 0)


@jax.jit
def _run(idx_flat, t2):
    k = functools.partial(
        pl.kernel,
        out_type=jax.ShapeDtypeStruct((R, DC), jnp.float32),
        mesh=plsc.VectorSubcoreMesh(core_axis_name="c", subcore_axis_name="s"),
        scratch_types=[
            pltpu.VMEM((BPW,), jnp.int32),
            pltpu.VMEM((G,), jnp.int32),
            pltpu.VMEM((G,), jnp.int32),
            pltpu.VMEM((G, DC), jnp.float32),
            pltpu.SemaphoreType.DMA,
            pltpu.SemaphoreType.DMA,
        ],
    )(_gather_kernel)
    return k(t2, idx_flat)


def kernel(prefix, embedding_weight):
    idx_flat = prefix.reshape(-1).astype(jnp.int32)
    t2 = embedding_weight.reshape(PRE_LEN * NCH, DC)
    out2 = _run(idx_flat, t2)
    return out2.reshape(prefix.shape[0], prefix.shape[1], D)


# speedup vs baseline: 1.1483x; 1.1483x over previous
import functools

import jax
import jax.numpy as jnp
from jax import lax
from jax.experimental import pallas as pl
from jax.experimental.pallas import tpu as pltpu
from jax.experimental.pallas import tpu_sc as plsc

PRE_LEN = 128
B_ROWS = 1024
D = 49152
NCH = 32
DC = D // NCH
R = B_ROWS * NCH
NW = 32
BPW = B_ROWS // NW
G = 16
NGRP = BPW // G
NIT = NCH * NGRP
NSLOT = 4


def _gather_kernel(t_hbm, idx_hbm, out_hbm, idx_full, *scr):
    gst = scr[0:NSLOT]
    ost = scr[NSLOT:2 * NSLOT]
    buf = scr[2 * NSLOT:3 * NSLOT]
    sem_g = scr[3 * NSLOT:4 * NSLOT]
    sem_s = scr[4 * NSLOT:5 * NSLOT]

    wid = lax.axis_index("s") * 2 + lax.axis_index("c")
    bs0 = wid * BPW
    pltpu.sync_copy(idx_hbm.at[pl.ds(bs0, BPW)], idx_full)
    lanes = lax.iota(jnp.int32, G)

    def fire_gather(it, b):
        c = lax.shift_right_logical(it, 1)
        grp = lax.bitwise_and(it, 1)
        pv = idx_full[pl.ds(grp * G, G)]
        gst[b][...] = pv * NCH + c
        ost[b][...] = (bs0 + grp * G + lanes) * NCH + c
        pltpu.make_async_copy(t_hbm.at[gst[b]], buf[b], sem_g[b]).start()

    for b in range(NSLOT):
        fire_gather(jnp.int32(b), b)

    def body(k, carry):
        for b in range(NSLOT):
            it = k * NSLOT + b
            pltpu.make_async_copy(t_hbm.at[gst[b]], buf[b], sem_g[b]).wait()
            pltpu.make_async_copy(buf[b], out_hbm.at[ost[b]],
                                  sem_s[b]).start()
            nxt = it + NSLOT

            @pl.when(nxt < NIT)
            def _():
                pltpu.make_async_copy(buf[b], out_hbm.at[ost[b]],
                                      sem_s[b]).wait()
                fire_gather(nxt, b)

        return carry

    lax.fori_loop(0, NIT // NSLOT, body, 0)

    for b in range(NSLOT):
        pltpu.make_async_copy(buf[b], out_hbm.at[ost[b]], sem_s[b]).wait()


@jax.jit
def _run(idx_flat, t2):
    k = functools.partial(
        pl.kernel,
        out_type=jax.ShapeDtypeStruct((R, DC), jnp.float32),
        mesh=plsc.VectorSubcoreMesh(core_axis_name="c", subcore_axis_name="s"),
        scratch_types=(
            [pltpu.VMEM((BPW,), jnp.int32)]
            + [pltpu.VMEM((G,), jnp.int32)] * NSLOT
            + [pltpu.VMEM((G,), jnp.int32)] * NSLOT
            + [pltpu.VMEM((G, DC), jnp.float32)] * NSLOT
            + [pltpu.SemaphoreType.DMA] * (2 * NSLOT)
        ),
    )(_gather_kernel)
    return k(t2, idx_flat)


def kernel(prefix, embedding_weight):
    idx_flat = prefix.reshape(-1).astype(jnp.int32)
    t2 = embedding_weight.reshape(PRE_LEN * NCH, DC)
    out2 = _run(idx_flat, t2)
    return out2.reshape(prefix.shape[0], prefix.shape[1], D)


# speedup vs baseline: 1.1963x; 1.0418x over previous
import functools

import jax
import jax.numpy as jnp
from jax import lax
from jax.experimental import pallas as pl
from jax.experimental.pallas import tpu as pltpu
from jax.experimental.pallas import tpu_sc as plsc

PRE_LEN = 128
B_ROWS = 1024
D = 49152
NCH = 64
DC = D // NCH
R = B_ROWS * NCH
NW = 32
BPW = B_ROWS // NW
G = 16
NGRP = BPW // G
NIT = NCH * NGRP
NS = 8
LA = 4


def _gather_kernel(t_hbm, idx_hbm, out_hbm, idx_full, *scr):
    gst = scr[0:NS]
    ost = scr[NS:2 * NS]
    buf = scr[2 * NS:3 * NS]
    sem_g = scr[3 * NS:4 * NS]
    sem_s = scr[4 * NS:5 * NS]

    wid = lax.axis_index("s") * 2 + lax.axis_index("c")
    bs0 = wid * BPW
    pltpu.sync_copy(idx_hbm.at[pl.ds(bs0, BPW)], idx_full)
    lanes = lax.iota(jnp.int32, G)

    def fire_gather(it, b):
        c = lax.shift_right_logical(it, 1)
        grp = lax.bitwise_and(it, 1)
        pv = idx_full[pl.ds(grp * G, G)]
        gst[b][...] = pv * NCH + c
        ost[b][...] = (bs0 + grp * G + lanes) * NCH + c
        pltpu.make_async_copy(t_hbm.at[gst[b]], buf[b], sem_g[b]).start()

    for j in range(LA):
        fire_gather(jnp.int32(j), j)

    def body(k, carry):
        for b in range(NS):
            j = k * NS + b
            pltpu.make_async_copy(t_hbm.at[gst[b]], buf[b], sem_g[b]).wait()
            pltpu.make_async_copy(buf[b], out_hbm.at[ost[b]],
                                  sem_s[b]).start()
            b2 = (b + LA) % NS
            nxt = j + LA

            @pl.when(jnp.logical_and(nxt < NIT, nxt >= NS))
            def _():
                pltpu.make_async_copy(buf[b2], out_hbm.at[ost[b2]],
                                      sem_s[b2]).wait()

            @pl.when(nxt < NIT)
            def _():
                fire_gather(nxt, b2)

        return carry

    lax.fori_loop(0, NIT // NS, body, 0)

    for b in range(NS):
        pltpu.make_async_copy(buf[b], out_hbm.at[ost[b]], sem_s[b]).wait()


@jax.jit
def _run(idx_flat, t2):
    k = functools.partial(
        pl.kernel,
        out_type=jax.ShapeDtypeStruct((R, DC), jnp.float32),
        mesh=plsc.VectorSubcoreMesh(core_axis_name="c", subcore_axis_name="s"),
        scratch_types=(
            [pltpu.VMEM((BPW,), jnp.int32)]
            + [pltpu.VMEM((G,), jnp.int32)] * NS
            + [pltpu.VMEM((G,), jnp.int32)] * NS
            + [pltpu.VMEM((G, DC), jnp.float32)] * NS
            + [pltpu.SemaphoreType.DMA] * (2 * NS)
        ),
    )(_gather_kernel)
    return k(t2, idx_flat)


def kernel(prefix, embedding_weight):
    idx_flat = prefix.reshape(-1).astype(jnp.int32)
    t2 = embedding_weight.reshape(PRE_LEN * NCH, DC)
    out2 = _run(idx_flat, t2)
    return out2.reshape(prefix.shape[0], prefix.shape[1], D)


# speedup vs baseline: 3.8666x; 3.2321x over previous
import functools

import jax
import jax.numpy as jnp
from jax.experimental import pallas as pl
from jax.experimental.pallas import tpu as pltpu

PRE_LEN = 128
B_ROWS = 1024
D = 49152
BR = 8
GRID = B_ROWS // BR


def _copy_kernel(idx_ref, t_ref, o_ref):
    i = pl.program_id(0)
    for r in range(BR):
        v = idx_ref[i * BR + r]
        o_ref[r, :] = t_ref[v, :]


@jax.jit
def _run(idx_flat, table):
    grid_spec = pltpu.PrefetchScalarGridSpec(
        num_scalar_prefetch=1,
        grid=(GRID,),
        in_specs=[pl.BlockSpec((PRE_LEN, D), lambda i, idx_ref: (0, 0))],
        out_specs=pl.BlockSpec((BR, D), lambda i, idx_ref: (i, 0)),
    )
    return pl.pallas_call(
        _copy_kernel,
        grid_spec=grid_spec,
        out_shape=jax.ShapeDtypeStruct((B_ROWS, D), jnp.float32),
    )(idx_flat, table)


def kernel(prefix, embedding_weight):
    idx_flat = prefix.reshape(-1).astype(jnp.int32)
    out2 = _run(idx_flat, embedding_weight)
    return out2.reshape(prefix.shape[0], prefix.shape[1], D)


# speedup vs baseline: 4.6163x; 1.1939x over previous
import functools

import jax
import jax.numpy as jnp
from jax.experimental import pallas as pl
from jax.experimental.pallas import tpu as pltpu

PRE_LEN = 128
B_ROWS = 1024
D = 49152
BR = 16
GRID = B_ROWS // BR


def _copy_kernel(idx_ref, t_ref, o_ref):
    i = pl.program_id(0)
    for r in range(BR):
        v = idx_ref[i * BR + r]
        o_ref[r, :] = t_ref[v, :]


@jax.jit
def _run(idx_flat, table):
    grid_spec = pltpu.PrefetchScalarGridSpec(
        num_scalar_prefetch=1,
        grid=(GRID,),
        in_specs=[pl.BlockSpec((PRE_LEN, D), lambda i, idx_ref: (0, 0))],
        out_specs=pl.BlockSpec((BR, D), lambda i, idx_ref: (i, 0)),
    )
    return pl.pallas_call(
        _copy_kernel,
        grid_spec=grid_spec,
        out_shape=jax.ShapeDtypeStruct((B_ROWS, D), jnp.float32),
    )(idx_flat, table)


def kernel(prefix, embedding_weight):
    idx_flat = prefix.reshape(-1).astype(jnp.int32)
    out2 = _run(idx_flat, embedding_weight)
    return out2.reshape(prefix.shape[0], prefix.shape[1], D)


# speedup vs baseline: 5.1268x; 1.1106x over previous
import functools

import jax
import jax.numpy as jnp
from jax.experimental import pallas as pl
from jax.experimental.pallas import tpu as pltpu

PRE_LEN = 128
B_ROWS = 1024
D = 49152
BR = 32
GRID = B_ROWS // BR


def _copy_kernel(idx_ref, t_ref, o_ref):
    i = pl.program_id(0)
    for r in range(BR):
        v = idx_ref[i * BR + r]
        o_ref[r, :] = t_ref[v, :]


@jax.jit
def _run(idx_flat, table):
    grid_spec = pltpu.PrefetchScalarGridSpec(
        num_scalar_prefetch=1,
        grid=(GRID,),
        in_specs=[pl.BlockSpec((PRE_LEN, D), lambda i, idx_ref: (0, 0))],
        out_specs=pl.BlockSpec((BR, D), lambda i, idx_ref: (i, 0)),
    )
    return pl.pallas_call(
        _copy_kernel,
        grid_spec=grid_spec,
        out_shape=jax.ShapeDtypeStruct((B_ROWS, D), jnp.float32),
    )(idx_flat, table)


def kernel(prefix, embedding_weight):
    idx_flat = prefix.reshape(-1).astype(jnp.int32)
    out2 = _run(idx_flat, embedding_weight)
    return out2.reshape(prefix.shape[0], prefix.shape[1], D)


# speedup vs baseline: 5.4489x; 1.0628x over previous
import functools

import jax
import jax.numpy as jnp
from jax.experimental import pallas as pl
from jax.experimental.pallas import tpu as pltpu

PRE_LEN = 128
B_ROWS = 1024
D = 49152
BR = 64
GRID = B_ROWS // BR


def _copy_kernel(idx_ref, t_ref, o_ref):
    i = pl.program_id(0)
    for r in range(BR):
        v = idx_ref[i * BR + r]
        o_ref[r, :] = t_ref[v, :]


@jax.jit
def _run(idx_flat, table):
    grid_spec = pltpu.PrefetchScalarGridSpec(
        num_scalar_prefetch=1,
        grid=(GRID,),
        in_specs=[pl.BlockSpec((PRE_LEN, D), lambda i, idx_ref: (0, 0))],
        out_specs=pl.BlockSpec((BR, D), lambda i, idx_ref: (i, 0)),
    )
    return pl.pallas_call(
        _copy_kernel,
        grid_spec=grid_spec,
        out_shape=jax.ShapeDtypeStruct((B_ROWS, D), jnp.float32),
    )(idx_flat, table)


def kernel(prefix, embedding_weight):
    idx_flat = prefix.reshape(-1).astype(jnp.int32)
    out2 = _run(idx_flat, embedding_weight)
    return out2.reshape(prefix.shape[0], prefix.shape[1], D)
